# baseline (speedup 1.0000x reference)
import jax
import jax.numpy as jnp
from jax import lax
from jax.experimental import pallas as pl
from jax.experimental.pallas import tpu as pltpu
from jax.experimental.pallas import tpu_sc as plsc

SMOOTHING_VALUE = 0.1
PAD_TOKEN_ID = 0
TRG_VOCAB_SIZE = 100000
CONFIDENCE_VALUE = 1.0 - SMOOTHING_VALUE
SMOOTH = SMOOTHING_VALUE / (TRG_VOCAB_SIZE - 2)

BATCH = 1024
V = TRG_VOCAB_SIZE

ROWS = 8
NBUF = 4
NSTEPS = BATCH // ROWS


def _fill_kernel(idx_ref, out_ref, buf, sem):
    g = pl.program_id(0)
    slot = jax.lax.rem(g, NBUF)

    @pl.when(g >= NBUF)
    def _():
        pltpu.make_async_copy(
            buf.at[slot], out_ref.at[pl.ds((g - NBUF) * ROWS, ROWS), :], sem.at[slot]
        ).wait()

    idx = idx_ref[pl.ds(g * ROWS, ROWS), :]
    base = jnp.where(idx == PAD_TOKEN_ID, jnp.float32(0.0), jnp.float32(SMOOTH))
    buf[slot] = jnp.broadcast_to(base, (ROWS, V))

    pltpu.make_async_copy(
        buf.at[slot], out_ref.at[pl.ds(g * ROWS, ROWS), :], sem.at[slot]
    ).start()

    @pl.when(g == NSTEPS - 1)
    def _():
        for k in range(NBUF):
            step = NSTEPS - NBUF + k
            s = step % NBUF
            pltpu.make_async_copy(
                buf.at[s], out_ref.at[pl.ds(step * ROWS, ROWS), :], sem.at[s]
            ).wait()


NW = 32
ROWS_PER_W = BATCH // NW
L = 16


def _scatter_body(idx_hbm, out_ref, idx_v, off_v, val_v, sem):
    nc = 2
    wid = lax.axis_index("s") * nc + lax.axis_index("c")
    row0 = wid * ROWS_PER_W

    pltpu.sync_copy(idx_hbm.at[pl.ds(row0, ROWS_PER_W)], idx_v)

    lanes = lax.broadcasted_iota(jnp.int32, (L,), 0)
    for h in range(ROWS_PER_W // L):
        idx16 = idx_v[pl.ds(h * L, L)]
        rows16 = row0 + h * L + lanes
        off_v[pl.ds(h * L, L)] = rows16 * V + idx16
        val_v[pl.ds(h * L, L)] = jnp.where(
            idx16 == PAD_TOKEN_ID, jnp.float32(0.0), jnp.float32(CONFIDENCE_VALUE)
        )

    pltpu.async_copy(val_v, out_ref.at[off_v], sem).wait()


@jax.jit
def kernel(trg_token_ids_batch):
    idx2d = trg_token_ids_batch.astype(jnp.int32)
    filled = pl.pallas_call(
        _fill_kernel,
        grid=(NSTEPS,),
        in_specs=[pl.BlockSpec(memory_space=pltpu.MemorySpace.VMEM)],
        out_specs=pl.BlockSpec(memory_space=pltpu.MemorySpace.HBM),
        out_shape=jax.ShapeDtypeStruct((BATCH, V), jnp.float32),
        scratch_shapes=[
            pltpu.VMEM((NBUF, ROWS, V), jnp.float32),
            pltpu.SemaphoreType.DMA((NBUF,)),
        ],
    )(idx2d)

    mesh = plsc.VectorSubcoreMesh(core_axis_name="c", subcore_axis_name="s")
    scatter = pl.kernel(
        _scatter_body,
        out_type=(),
        mesh=mesh,
        scratch_types=[
            pltpu.VMEM((ROWS_PER_W,), jnp.int32),
            pltpu.VMEM((ROWS_PER_W,), jnp.int32),
            pltpu.VMEM((ROWS_PER_W,), jnp.float32),
            pltpu.SemaphoreType.DMA,
        ],
    )
    flat_ref = jax.new_ref(filled.reshape((BATCH * V,)))
    scatter(idx2d.reshape((BATCH,)), flat_ref)
    return flat_ref[...].reshape((BATCH, V))


# speedup vs baseline: 1.5919x; 1.5919x over previous
import jax
import jax.numpy as jnp
from jax import lax
from jax.experimental import pallas as pl
from jax.experimental.pallas import tpu as pltpu
from jax.experimental.pallas import tpu_sc as plsc

SMOOTHING_VALUE = 0.1
PAD_TOKEN_ID = 0
TRG_VOCAB_SIZE = 100000
CONFIDENCE_VALUE = 1.0 - SMOOTHING_VALUE
SMOOTH = SMOOTHING_VALUE / (TRG_VOCAB_SIZE - 2)

BATCH = 1024
V = TRG_VOCAB_SIZE
NW = 32
ROWS_PER_W = BATCH // NW
GROUPS_PER_W = ROWS_PER_W // 8
NTILES = V // 128
TAIL0 = NTILES * 128
L = 16


def _sc_body(idx_hbm, smooth_hbm, out_hbm, idx_v, padpatt_v,
             smooth_sh, patt_sh, sem_fill, sem_patt):
    nc = 2
    sid = lax.axis_index("s")
    wid = sid * nc + lax.axis_index("c")
    row0 = wid * ROWS_PER_W

    pltpu.sync_copy(idx_hbm.at[pl.ds(row0, ROWS_PER_W)], idx_v)

    @pl.when(sid == 0)
    def _():
        pltpu.sync_copy(smooth_hbm, smooth_sh)

    plsc.subcore_barrier()

    halves = [idx_v[pl.ds(0, L)], idx_v[pl.ds(L, L)]]
    s = [
        jnp.squeeze(lax.slice(halves[r // L], (r % L,), (r % L + 1,)))
        for r in range(ROWS_PER_W)
    ]

    smooth16 = jnp.full((L,), SMOOTH, dtype=jnp.float32)
    zero16 = jnp.zeros((L,), dtype=jnp.float32)

    pad_any = []
    for k in range(GROUPS_PER_W):
        sg = s[k * 8:(k + 1) * 8]
        p = sg[0] == PAD_TOKEN_ID
        for j in range(1, 8):
            p = jnp.logical_or(p, sg[j] == PAD_TOKEN_ID)
        pad_any.append(p)

    for k in range(GROUPS_PER_W):
        g = wid * GROUPS_PER_W + k
        sg = s[k * 8:(k + 1) * 8]

        def smooth_fill(g=g):
            pltpu.make_async_copy(
                smooth_sh, out_hbm.at[pl.ds(g * 8, 8), :], sem_fill
            ).start()

        def pattern_fill(g=g, sg=sg):
            for j in range(8):
                row_is_pad = jnp.full((L,), sg[j], jnp.int32) == PAD_TOKEN_ID
                val16 = jnp.where(row_is_pad, zero16, smooth16)
                for u in range(8):
                    padpatt_v[pl.ds(u * L, L)] = val16
                pltpu.sync_copy(padpatt_v, patt_sh.at[sid * 8 + j, :])

            def fire(c, _):
                pltpu.make_async_copy(
                    patt_sh.at[pl.ds(sid * 8, 8), :],
                    out_hbm.at[pl.ds(g * 8, 8),
                               pl.ds(pl.multiple_of(c * 128, 128), 128)],
                    sem_patt,
                ).start()
                return 0

            lax.fori_loop(0, NTILES, fire, 0)

            def drain(c, _):
                pltpu.make_async_copy(
                    patt_sh.at[pl.ds(sid * 8, 8), :],
                    out_hbm.at[pl.ds(g * 8, 8),
                               pl.ds(pl.multiple_of(c * 128, 128), 128)],
                    sem_patt,
                ).wait()
                return 0

            lax.fori_loop(0, NTILES, drain, 0)

        lax.cond(pad_any[k], pattern_fill, smooth_fill)

    for k in range(GROUPS_PER_W):
        g = wid * GROUPS_PER_W + k

        def wait_smooth(g=g):
            pltpu.make_async_copy(
                smooth_sh, out_hbm.at[pl.ds(g * 8, 8), :], sem_fill
            ).wait()

        lax.cond(pad_any[k], lambda: None, wait_smooth)


def _reconstruct_block(idx, col0):
    cols = col0 + lax.broadcasted_iota(jnp.int32, (8, 128), 1)
    val = jnp.where(cols == idx, jnp.float32(CONFIDENCE_VALUE),
                    jnp.float32(SMOOTH))
    return jnp.where(idx == PAD_TOKEN_ID, jnp.float32(0.0), val)


def _conf_kernel(idxp_ref, idx_ref, alias_ref, out_ref):
    del alias_ref
    i = pl.program_id(0)
    cb = idxp_ref[i] // 128
    out_ref[...] = _reconstruct_block(idx_ref[...], cb * 128)


def _tail_kernel(idx_ref, alias_ref, out_ref):
    del alias_ref
    out_ref[...] = _reconstruct_block(idx_ref[...], TAIL0)


@jax.jit
def kernel(trg_token_ids_batch):
    idx2d = trg_token_ids_batch.astype(jnp.int32)
    idx = idx2d.reshape((BATCH,))
    smooth_block = jnp.full((8, V), SMOOTH, dtype=jnp.float32)

    mesh = plsc.VectorSubcoreMesh(core_axis_name="c", subcore_axis_name="s")
    sc_fill = pl.kernel(
        _sc_body,
        out_type=jax.ShapeDtypeStruct((BATCH, V), jnp.float32),
        mesh=mesh,
        scratch_types=[
            pltpu.VMEM((ROWS_PER_W,), jnp.int32),
            pltpu.VMEM((128,), jnp.float32),
            pltpu.MemorySpace.VMEM_SHARED((8, V), jnp.float32),
            pltpu.MemorySpace.VMEM_SHARED((128, 128), jnp.float32),
            pltpu.SemaphoreType.DMA,
            pltpu.SemaphoreType.DMA,
        ],
    )
    filled = sc_fill(idx, smooth_block)

    scattered = pl.pallas_call(
        _conf_kernel,
        grid_spec=pltpu.PrefetchScalarGridSpec(
            num_scalar_prefetch=1,
            grid=(BATCH,),
            in_specs=[
                pl.BlockSpec((8, 1), lambda i, idxp: (i // 8, 0)),
                pl.BlockSpec(memory_space=pltpu.MemorySpace.HBM),
            ],
            out_specs=pl.BlockSpec((8, 128),
                                   lambda i, idxp: (i // 8, idxp[i] // 128)),
        ),
        out_shape=jax.ShapeDtypeStruct((BATCH, V), jnp.float32),
        input_output_aliases={2: 0},
    )(idx, idx2d, filled)

    return pl.pallas_call(
        _tail_kernel,
        grid=(BATCH // 8,),
        in_specs=[
            pl.BlockSpec((8, 1), lambda g: (g, 0)),
            pl.BlockSpec(memory_space=pltpu.MemorySpace.HBM),
        ],
        out_specs=pl.BlockSpec((8, 128), lambda g: (g, NTILES)),
        out_shape=jax.ShapeDtypeStruct((BATCH, V), jnp.float32),
        input_output_aliases={1: 0},
    )(idx2d, scattered)


# speedup vs baseline: 2.2895x; 1.4382x over previous
import jax
import jax.numpy as jnp
from jax import lax
from jax.experimental import pallas as pl
from jax.experimental.pallas import tpu as pltpu
from jax.experimental.pallas import tpu_sc as plsc

SMOOTHING_VALUE = 0.1
PAD_TOKEN_ID = 0
TRG_VOCAB_SIZE = 100000
CONFIDENCE_VALUE = 1.0 - SMOOTHING_VALUE
SMOOTH = SMOOTHING_VALUE / (TRG_VOCAB_SIZE - 2)

BATCH = 1024
V = TRG_VOCAB_SIZE
NW = 32
ROWS_PER_W = BATCH // NW
GROUPS_PER_W = ROWS_PER_W // 8
NTILES = V // 128
TAIL0 = NTILES * 128
L = 16


def _sc_body(idx_hbm, smooth_hbm, out_hbm, idx_v, padpatt_v,
             smooth_sh, patt_sh, sem_fill, sem_patt):
    nc = 2
    sid = lax.axis_index("s")
    wid = sid * nc + lax.axis_index("c")
    row0 = wid * ROWS_PER_W

    pltpu.sync_copy(idx_hbm.at[pl.ds(row0, ROWS_PER_W)], idx_v)

    @pl.when(sid == 0)
    def _():
        pltpu.sync_copy(smooth_hbm, smooth_sh)

    plsc.subcore_barrier()

    halves = [idx_v[pl.ds(0, L)], idx_v[pl.ds(L, L)]]
    s = [
        jnp.squeeze(lax.slice(halves[r // L], (r % L,), (r % L + 1,)))
        for r in range(ROWS_PER_W)
    ]

    smooth16 = jnp.full((L,), SMOOTH, dtype=jnp.float32)
    zero16 = jnp.zeros((L,), dtype=jnp.float32)

    pad_any = []
    for k in range(GROUPS_PER_W):
        sg = s[k * 8:(k + 1) * 8]
        p = sg[0] == PAD_TOKEN_ID
        for j in range(1, 8):
            p = jnp.logical_or(p, sg[j] == PAD_TOKEN_ID)
        pad_any.append(p)

    for k in range(GROUPS_PER_W):
        g = wid * GROUPS_PER_W + k
        sg = s[k * 8:(k + 1) * 8]

        def smooth_fill(g=g):
            pltpu.make_async_copy(
                smooth_sh, out_hbm.at[pl.ds(g * 8, 8), :], sem_fill
            ).start()

        def pattern_fill(g=g, sg=sg):
            for j in range(8):
                row_is_pad = jnp.full((L,), sg[j], jnp.int32) == PAD_TOKEN_ID
                val16 = jnp.where(row_is_pad, zero16, smooth16)
                for u in range(8):
                    padpatt_v[pl.ds(u * L, L)] = val16
                pltpu.sync_copy(padpatt_v, patt_sh.at[sid * 8 + j, :])

            def fire(c, _):
                pltpu.make_async_copy(
                    patt_sh.at[pl.ds(sid * 8, 8), :],
                    out_hbm.at[pl.ds(g * 8, 8),
                               pl.ds(pl.multiple_of(c * 128, 128), 128)],
                    sem_patt,
                ).start()
                return 0

            lax.fori_loop(0, NTILES, fire, 0)

            def drain(c, _):
                pltpu.make_async_copy(
                    patt_sh.at[pl.ds(sid * 8, 8), :],
                    out_hbm.at[pl.ds(g * 8, 8),
                               pl.ds(pl.multiple_of(c * 128, 128), 128)],
                    sem_patt,
                ).wait()
                return 0

            lax.fori_loop(0, NTILES, drain, 0)

        lax.cond(pad_any[k], pattern_fill, smooth_fill)

    for k in range(GROUPS_PER_W):
        g = wid * GROUPS_PER_W + k

        def wait_smooth(g=g):
            pltpu.make_async_copy(
                smooth_sh, out_hbm.at[pl.ds(g * 8, 8), :], sem_fill
            ).wait()

        lax.cond(pad_any[k], lambda: None, wait_smooth)

    lanes = lax.broadcasted_iota(jnp.int32, (L,), 0)

    for r in range(ROWS_PER_W):
        def conf_patch(r=r):
            k = r // 8
            g = wid * GROUPS_PER_W + k
            srow = 128 + sid * 8
            c0 = pl.multiple_of((s[r] // 128) * 128, 128)
            pltpu.sync_copy(out_hbm.at[pl.ds(g * 8, 8), pl.ds(c0, 128)],
                            patt_sh.at[pl.ds(srow, 8), :])
            pos = s[r] - c0
            u0 = pl.multiple_of((pos // L) * L, L)
            lane = pos - u0
            chunk = jnp.where(lanes == lane,
                              jnp.float32(CONFIDENCE_VALUE), smooth16)
            for u in range(8):
                padpatt_v[pl.ds(u * L, L)] = smooth16
            padpatt_v[pl.ds(u0, L)] = chunk
            pltpu.sync_copy(padpatt_v, patt_sh.at[srow + (r % 8), :])
            pltpu.sync_copy(patt_sh.at[pl.ds(srow, 8), :],
                            out_hbm.at[pl.ds(g * 8, 8), pl.ds(c0, 128)])

        do = jnp.logical_and(s[r] != PAD_TOKEN_ID, s[r] < TAIL0)
        lax.cond(do, conf_patch, lambda: None)


def _reconstruct_block(idx, col0):
    cols = col0 + lax.broadcasted_iota(jnp.int32, (8, 128), 1)
    val = jnp.where(cols == idx, jnp.float32(CONFIDENCE_VALUE),
                    jnp.float32(SMOOTH))
    return jnp.where(idx == PAD_TOKEN_ID, jnp.float32(0.0), val)


def _tail_kernel(idx_ref, alias_ref, out_ref):
    del alias_ref
    out_ref[...] = _reconstruct_block(idx_ref[...], TAIL0)


@jax.jit
def kernel(trg_token_ids_batch):
    idx2d = trg_token_ids_batch.astype(jnp.int32)
    idx = idx2d.reshape((BATCH,))
    smooth_block = jnp.full((8, V), SMOOTH, dtype=jnp.float32)

    mesh = plsc.VectorSubcoreMesh(core_axis_name="c", subcore_axis_name="s")
    sc_fill = pl.kernel(
        _sc_body,
        out_type=jax.ShapeDtypeStruct((BATCH, V), jnp.float32),
        mesh=mesh,
        scratch_types=[
            pltpu.VMEM((ROWS_PER_W,), jnp.int32),
            pltpu.VMEM((128,), jnp.float32),
            pltpu.MemorySpace.VMEM_SHARED((8, V), jnp.float32),
            pltpu.MemorySpace.VMEM_SHARED((256, 128), jnp.float32),
            pltpu.SemaphoreType.DMA,
            pltpu.SemaphoreType.DMA,
        ],
    )
    filled = sc_fill(idx, smooth_block)

    return pl.pallas_call(
        _tail_kernel,
        grid=(BATCH // 8,),
        in_specs=[
            pl.BlockSpec((8, 1), lambda g: (g, 0)),
            pl.BlockSpec(memory_space=pltpu.MemorySpace.HBM),
        ],
        out_specs=pl.BlockSpec((8, 128), lambda g: (g, NTILES)),
        out_shape=jax.ShapeDtypeStruct((BATCH, V), jnp.float32),
        input_output_aliases={1: 0},
    )(idx2d, filled)


# speedup vs baseline: 2.5007x; 1.0922x over previous
import jax
import jax.numpy as jnp
from jax import lax
from jax.experimental import pallas as pl
from jax.experimental.pallas import tpu as pltpu
from jax.experimental.pallas import tpu_sc as plsc

SMOOTHING_VALUE = 0.1
PAD_TOKEN_ID = 0
TRG_VOCAB_SIZE = 100000
CONFIDENCE_VALUE = 1.0 - SMOOTHING_VALUE
SMOOTH = SMOOTHING_VALUE / (TRG_VOCAB_SIZE - 2)

BATCH = 1024
V = TRG_VOCAB_SIZE
NW = 32
ROWS_PER_W = BATCH // NW
GROUPS_PER_W = ROWS_PER_W // 8
NTILES = V // 128
TAIL0 = NTILES * 128
L = 16


def _sc_body(idx_hbm, smooth_hbm, out_hbm, idx_v, padpatt_v,
             smooth_sh, patt_sh, sem_fill, sem_patt):
    nc = 2
    sid = lax.axis_index("s")
    wid = sid * nc + lax.axis_index("c")
    row0 = wid * ROWS_PER_W

    pltpu.sync_copy(idx_hbm.at[pl.ds(row0, ROWS_PER_W)], idx_v)

    @pl.when(sid == 0)
    def _():
        pltpu.sync_copy(smooth_hbm, smooth_sh)

    plsc.subcore_barrier()

    halves = [idx_v[pl.ds(0, L)], idx_v[pl.ds(L, L)]]
    s = [
        jnp.squeeze(lax.slice(halves[r // L], (r % L,), (r % L + 1,)))
        for r in range(ROWS_PER_W)
    ]

    smooth16 = jnp.full((L,), SMOOTH, dtype=jnp.float32)
    zero16 = jnp.zeros((L,), dtype=jnp.float32)

    pad_any = []
    for k in range(GROUPS_PER_W):
        sg = s[k * 8:(k + 1) * 8]
        p = sg[0] == PAD_TOKEN_ID
        for j in range(1, 8):
            p = jnp.logical_or(p, sg[j] == PAD_TOKEN_ID)
        pad_any.append(p)

    for k in range(GROUPS_PER_W):
        g = wid * GROUPS_PER_W + k
        sg = s[k * 8:(k + 1) * 8]

        def smooth_fill(g=g):
            pltpu.make_async_copy(
                smooth_sh, out_hbm.at[pl.ds(g * 8, 8), :], sem_fill
            ).start()

        def pattern_fill(g=g, sg=sg):
            for j in range(8):
                row_is_pad = jnp.full((L,), sg[j], jnp.int32) == PAD_TOKEN_ID
                val16 = jnp.where(row_is_pad, zero16, smooth16)
                for u in range(8):
                    padpatt_v[pl.ds(u * L, L)] = val16
                pltpu.sync_copy(padpatt_v, patt_sh.at[sid * 8 + j, :])

            def fire(c, _):
                pltpu.make_async_copy(
                    patt_sh.at[pl.ds(sid * 8, 8), :],
                    out_hbm.at[pl.ds(g * 8, 8),
                               pl.ds(pl.multiple_of(c * 128, 128), 128)],
                    sem_patt,
                ).start()
                return 0

            lax.fori_loop(0, NTILES, fire, 0)

            def drain(c, _):
                pltpu.make_async_copy(
                    patt_sh.at[pl.ds(sid * 8, 8), :],
                    out_hbm.at[pl.ds(g * 8, 8),
                               pl.ds(pl.multiple_of(c * 128, 128), 128)],
                    sem_patt,
                ).wait()
                return 0

            lax.fori_loop(0, NTILES, drain, 0)

        lax.cond(pad_any[k], pattern_fill, smooth_fill)

    for k in range(GROUPS_PER_W):
        g = wid * GROUPS_PER_W + k

        def wait_smooth(g=g):
            pltpu.make_async_copy(
                smooth_sh, out_hbm.at[pl.ds(g * 8, 8), :], sem_fill
            ).wait()

        lax.cond(pad_any[k], lambda: None, wait_smooth)

    lanes = lax.broadcasted_iota(jnp.int32, (L,), 0)

    for r in range(ROWS_PER_W):
        def conf_patch(r=r):
            k = r // 8
            g = wid * GROUPS_PER_W + k
            srow = 128 + sid * 8
            c0 = pl.multiple_of((s[r] // 128) * 128, 128)
            pltpu.sync_copy(out_hbm.at[pl.ds(g * 8, 8), pl.ds(c0, 128)],
                            patt_sh.at[pl.ds(srow, 8), :])
            pos = s[r] - c0
            u0 = pl.multiple_of((pos // L) * L, L)
            lane = pos - u0
            chunk = jnp.where(lanes == lane,
                              jnp.float32(CONFIDENCE_VALUE), smooth16)
            for u in range(8):
                padpatt_v[pl.ds(u * L, L)] = smooth16
            padpatt_v[pl.ds(u0, L)] = chunk
            pltpu.sync_copy(padpatt_v, patt_sh.at[srow + (r % 8), :])
            pltpu.sync_copy(patt_sh.at[pl.ds(srow, 8), :],
                            out_hbm.at[pl.ds(g * 8, 8), pl.ds(c0, 128)])

        do = jnp.logical_and(s[r] != PAD_TOKEN_ID, s[r] < TAIL0)
        lax.cond(do, conf_patch, lambda: None)


def _reconstruct_block(idx, col0):
    cols = col0 + lax.broadcasted_iota(jnp.int32, (idx.shape[0], 128), 1)
    val = jnp.where(cols == idx, jnp.float32(CONFIDENCE_VALUE),
                    jnp.float32(SMOOTH))
    return jnp.where(idx == PAD_TOKEN_ID, jnp.float32(0.0), val)


def _tail_kernel(idx_ref, alias_ref, out_ref):
    del alias_ref
    out_ref[...] = _reconstruct_block(idx_ref[...], TAIL0)


@jax.jit
def kernel(trg_token_ids_batch):
    idx2d = trg_token_ids_batch.astype(jnp.int32)
    idx = idx2d.reshape((BATCH,))
    smooth_block = jnp.full((8, V), SMOOTH, dtype=jnp.float32)

    mesh = plsc.VectorSubcoreMesh(core_axis_name="c", subcore_axis_name="s")
    sc_fill = pl.kernel(
        _sc_body,
        out_type=jax.ShapeDtypeStruct((BATCH, V), jnp.float32),
        mesh=mesh,
        scratch_types=[
            pltpu.VMEM((ROWS_PER_W,), jnp.int32),
            pltpu.VMEM((128,), jnp.float32),
            pltpu.MemorySpace.VMEM_SHARED((8, V), jnp.float32),
            pltpu.MemorySpace.VMEM_SHARED((256, 128), jnp.float32),
            pltpu.SemaphoreType.DMA,
            pltpu.SemaphoreType.DMA,
        ],
    )
    filled = sc_fill(idx, smooth_block)

    return pl.pallas_call(
        _tail_kernel,
        grid=(1,),
        in_specs=[
            pl.BlockSpec((BATCH, 1), lambda g: (0, 0)),
            pl.BlockSpec(memory_space=pltpu.MemorySpace.HBM),
        ],
        out_specs=pl.BlockSpec((BATCH, 128), lambda g: (0, NTILES)),
        out_shape=jax.ShapeDtypeStruct((BATCH, V), jnp.float32),
        input_output_aliases={1: 0},
    )(idx2d, filled)
